# baseline (speedup 1.0000x reference)
import functools

import jax
import jax.numpy as jnp
from jax import lax
from jax.experimental import pallas as pl
from jax.experimental.pallas import tpu as pltpu
from jax.experimental.pallas import tpu_sc as plsc

VOCAB = 1000000
HID = 64
PAD = 0
SCALE = float(HID) ** 0.5

NC, NS, LANES = 2, 16, 16
NW = NC * NS
B_TOTAL = 4096 * 200
PER_W = B_TOTAL // NW
CHUNK = 400
NBUF = 4
N_CHUNKS = PER_W // CHUNK
assert N_CHUNKS % NBUF == 0


def _body(idx_hbm, table_hbm, out_hbm, idx_all, r0, r1, r2, r3,
          gs0, gs1, gs2, gs3, ss0, ss1, ss2, ss3):
  rbufs = [r0, r1, r2, r3]
  gsems = [gs0, gs1, gs2, gs3]
  ssems = [ss0, ss1, ss2, ss3]

  wid = lax.axis_index("s") * NC + lax.axis_index("c")
  base = wid * PER_W

  pltpu.sync_copy(idx_hbm.at[pl.ds(base, PER_W)], idx_all)

  def start_gather(chunk, b):
    pltpu.async_copy(
        table_hbm.at[idx_all.at[pl.ds(chunk * CHUNK, CHUNK)]],
        rbufs[b], gsems[b])

  def scale_chunk(chunk, b):
    rows = rbufs[b]
    dnums = lax.GatherDimensionNumbers(
        offset_dims=(), collapsed_slice_dims=(0,), start_index_map=(0,))

    def group(g, carry):
      idxv = idx_all[pl.ds(chunk * CHUNK + g * LANES, LANES)]
      mv = jnp.where(idxv == PAD,
                     jnp.zeros((LANES,), jnp.float32),
                     jnp.full((LANES,), SCALE, jnp.float32))
      for j in range(LANES):
        spl = lax.gather(mv, jnp.full((LANES, 1), j, jnp.int32), dnums, (1,),
                         mode=lax.GatherScatterMode.PROMISE_IN_BOUNDS)
        r = g * LANES + j
        for c in range(HID // LANES):
          rows[r, pl.ds(c * LANES, LANES)] = rows[r, pl.ds(c * LANES, LANES)] * spl
      return carry

    lax.fori_loop(0, CHUNK // LANES, group, 0)

  for b in range(NBUF):
    start_gather(b, b)

  def outer(s, carry):
    for b in range(NBUF):
      chunk = s * NBUF + b
      pltpu.make_async_copy(
          table_hbm.at[idx_all.at[pl.ds(chunk * CHUNK, CHUNK)]],
          rbufs[b], gsems[b]).wait()
      scale_chunk(chunk, b)
      pltpu.async_copy(
          rbufs[b], out_hbm.at[pl.ds(base + chunk * CHUNK, CHUNK)], ssems[b])
      nxt = chunk + NBUF

      @pl.when(nxt < N_CHUNKS)
      def _():
        pltpu.make_async_copy(
            rbufs[b], out_hbm.at[pl.ds(base + chunk * CHUNK, CHUNK)],
            ssems[b]).wait()
        start_gather(nxt, b)

    return carry

  lax.fori_loop(0, N_CHUNKS // NBUF, outer, 0)

  for b in range(NBUF):
    chunk = N_CHUNKS - NBUF + b
    pltpu.make_async_copy(
        rbufs[b], out_hbm.at[pl.ds(base + chunk * CHUNK, CHUNK)],
        ssems[b]).wait()


@functools.partial(jax.jit, static_argnames=())
def _run(idx_flat, table):
  mesh = plsc.VectorSubcoreMesh(core_axis_name="c", subcore_axis_name="s")
  k = pl.kernel(
      _body,
      out_type=jax.ShapeDtypeStruct((B_TOTAL, HID), jnp.float32),
      mesh=mesh,
      scratch_types=(
          [pltpu.VMEM((PER_W,), jnp.int32)]
          + [pltpu.VMEM((CHUNK, HID), jnp.float32) for _ in range(NBUF)]
          + [pltpu.SemaphoreType.DMA for _ in range(2 * NBUF)]
      ),
      compiler_params=pltpu.CompilerParams(use_tc_tiling_on_sc=False),
  )
  return k(idx_flat, table)


def kernel(inputs, shared_weights):
  idx_flat = inputs.reshape(-1).astype(jnp.int32)
  out = _run(idx_flat, shared_weights)
  return out.reshape(inputs.shape + (HID,))
